# baseline (speedup 1.0000x reference)
import math

import numpy as np
import jax
import jax.numpy as jnp
from jax import lax
from jax.experimental import pallas as pl

NBINS = 36
Q = 300
BATCH = 4
TOPK = 8
ALPHA_ITERS = 24

F = np.float32
NEG_INF = F(-np.inf)
POS_INF = F(np.inf)
TWO_PI = F(2.0 * math.pi)
INV_BINSZ = F(NBINS / (2.0 * math.pi))
HALF_PI = F(math.pi / 2.0)
ALPHA_LO = F(math.pi / NBINS)
QF = F(Q)
INV_QF = F(1.0 / Q)
ZERO = F(0.0)
ONE = F(1.0)
NEG1 = F(-1.0)
HALF = F(0.5)
TWO = F(2.0)
BIG_NEG = F(-1e30)
EPS = F(1e-12)

_F32 = jnp.float32
_HI = lax.Precision.HIGHEST


def _eye_f32(n):
    r = lax.broadcasted_iota(jnp.int32, (n, n), 0)
    c = lax.broadcasted_iota(jnp.int32, (n, n), 1)
    return (r == c).astype(_F32)


def _tp(x, eye):
    return lax.dot_general(x, eye, (((0,), (0,)), ((), ())),
                           preferred_element_type=_F32, precision=_HI)


def _circ(a, b):
    d = jnp.abs(a - b)
    return jnp.minimum(jnp.minimum(d, TWO_PI - d), HALF_PI)


def _ogqc_kernel(qf_ref, pa_ref, w_ref, coop_out, comp_out):
    e300 = _eye_f32(Q)
    e36 = _eye_f32(NBINS)

    ii = lax.broadcasted_iota(jnp.int32, (Q, Q), 0)
    jj = lax.broadcasted_iota(jnp.int32, (Q, Q), 1)
    lower = ii > jj
    row36_f = lax.broadcasted_iota(jnp.int32, (NBINS, 1), 0).astype(_F32)
    block_a = jnp.concatenate(
        [jnp.full((a + 1, 1), F(a), _F32) for a in range(NBINS)], axis=0)
    block_r = jnp.concatenate(
        [row36_f[0:a + 1, :] for a in range(NBINS)], axis=0)
    pad36 = jnp.full((NBINS, 1), ZERO, _F32)
    lane_q = lax.broadcasted_iota(jnp.int32, (1, Q), 1).astype(_F32)
    lane_36f = lax.broadcasted_iota(jnp.int32, (1, NBINS), 1).astype(_F32)
    aa36 = lax.broadcasted_iota(jnp.int32, (NBINS, NBINS), 0)
    cc36 = lax.broadcasted_iota(jnp.int32, (NBINS, NBINS), 1)
    ut36 = aa36 <= cc36
    offdiag36 = aa36 != cc36

    w_mat = w_ref[...]

    total_coop = jnp.full((), ZERO, _F32)
    total_comp = jnp.full((), ZERO, _F32)
    valid = jnp.full((), ZERO, _F32)

    for b in range(BATCH):
        pa = pa_ref[b]
        x_col = pa[:, 0:1]
        y_col = pa[:, 1:2]
        ang_col = jnp.arctan2(y_col, x_col)
        ang_col = jnp.where(ang_col < ZERO, ang_col + TWO_PI, ang_col)
        bins_f_col = jnp.clip(jnp.floor(ang_col * INV_BINSZ), ZERO,
                              F(NBINS - 1))
        bins_f_row = _tp(bins_f_col, e300)
        ang_row = _tp(ang_col, e300)

        onehot = (bins_f_col == lane_36f).astype(_F32)
        fused = qf_ref[b] + lax.dot_general(
            onehot, w_mat, (((1,), (0,)), ((), ())),
            preferred_element_type=_F32, precision=_HI)
        nrm = jnp.sqrt(jnp.sum(fused * fused, axis=1, keepdims=True))
        nq = fused / jnp.maximum(nrm, EPS)
        sim = lax.dot_general(nq, nq, (((1,), (1,)), ((), ())),
                              preferred_element_type=_F32, precision=_HI)

        sim_lo = jnp.where(lower, sim, NEG_INF)

        bound = jnp.full((), POS_INF, _F32)
        cum = jnp.full((), ZERO, _F32)
        kth = jnp.full((), NEG_INF, _F32)
        for _ in range(TOPK // 2):
            act = cum < F(TOPK // 2)
            m = jnp.max(jnp.where(sim_lo < bound, sim_lo, NEG_INF))
            cnt = jnp.sum((sim_lo == m).astype(_F32))
            kth = jnp.where(act, m, kth)
            cum = cum + jnp.where(act, cnt, ZERO)
            bound = jnp.where(act, m, bound)

        m1_rows = []
        for c in range(NBINS):
            v = jnp.where(bins_f_col == F(c), sim_lo, NEG_INF)
            m1_rows.append(jnp.max(v, axis=0, keepdims=True))
        m1t = jnp.concatenate(m1_rows, axis=0)

        nh_cols = []
        for a in range(NBINS):
            amask = bins_f_row == F(a)
            nh_cols.append(jnp.max(jnp.where(amask, m1t, NEG_INF), axis=1,
                                   keepdims=True))
        nhalf = jnp.maximum(jnp.concatenate(nh_cols, axis=1), BIG_NEG)
        nsym = jnp.maximum(nhalf, _tp(nhalf, e36))

        ch_cols = []
        for a in range(NBINS):
            cond = (bins_f_row == F(a)) & (m1t == nsym[:, a:a + 1])
            ch_cols.append(jnp.max(jnp.where(cond, lane_q, NEG1), axis=1,
                                   keepdims=True))
        codehalf = jnp.concatenate(ch_cols, axis=1)
        sel_i = jnp.maximum(codehalf, _tp(codehalf, e36))

        b1 = lax.dot_general(onehot, nsym, (((1,), (0,)), ((), ())),
                             preferred_element_type=_F32, precision=_HI)
        nsym_bcast = lax.dot_general(b1, onehot, (((1,), (1,)), ((), ())),
                                     preferred_element_type=_F32,
                                     precision=_HI)
        emat = (sim == nsym_bcast).astype(_F32)
        si_all = jnp.concatenate(
            [sel_i[0:a + 1, a:a + 1] for a in range(NBINS)], axis=0)
        oh_bool = si_all == lane_q
        oh_all = oh_bool.astype(_F32)
        r_e = lax.dot_general(oh_all, emat, (((1,), (0,)), ((), ())),
                              preferred_element_type=_F32)
        b_sel = jnp.sum(jnp.where(oh_bool, bins_f_row, ZERO), axis=1,
                        keepdims=True)
        ai_all = jnp.sum(jnp.where(oh_bool, ang_row, ZERO), axis=1,
                         keepdims=True)
        jbin_all = jnp.where(b_sel == block_a, block_r, block_a)
        maskj = ((bins_f_row == jbin_all) & (lane_q > si_all)
                 & (r_e > HALF))
        sj_all = jnp.max(jnp.where(maskj, lane_q, NEG1), axis=1,
                         keepdims=True)
        aj_all = jnp.sum(jnp.where(sj_all == lane_q, ang_row, ZERO), axis=1,
                         keepdims=True)
        ai_cols = []
        aj_cols = []
        off = 0
        for a in range(NBINS):
            h = a + 1
            if a < NBINS - 1:
                ai_cols.append(jnp.concatenate(
                    [ai_all[off:off + h, :], pad36[0:NBINS - h, :]], axis=0))
                aj_cols.append(jnp.concatenate(
                    [aj_all[off:off + h, :], pad36[0:NBINS - h, :]], axis=0))
            else:
                ai_cols.append(ai_all[off:off + h, :])
                aj_cols.append(aj_all[off:off + h, :])
            off += h
        ai = jnp.concatenate(ai_cols, axis=1)
        aj = jnp.concatenate(aj_cols, axis=1)

        hist_row = jnp.sum(onehot, axis=0, keepdims=True)
        hist_col = lax.dot_general(e36, hist_row, (((1,), (1,)), ((), ())),
                                   preferred_element_type=_F32, precision=_HI)
        present = ((hist_col >= ONE) & (hist_row >= ONE)
                   & (offdiag36 | (hist_col >= TWO)))
        final_mask = present & ut36 & (nsym >= kth)
        n = jnp.sum(final_mask.astype(_F32))

        pd = _circ(ai, aj)
        pdm = jnp.where(final_mask, pd, POS_INF)
        target = jnp.floor(jnp.maximum(n - ONE, ZERO) * HALF) + ONE
        bound = jnp.full((), NEG_INF, _F32)
        cum = jnp.full((), ZERO, _F32)
        alpha = jnp.full((), POS_INF, _F32)
        for _ in range(ALPHA_ITERS):
            act = cum < target
            m = jnp.min(jnp.where(pdm > bound, pdm, POS_INF))
            cnt = jnp.sum((pdm == m).astype(_F32))
            alpha = jnp.where(act, m, alpha)
            cum = cum + jnp.where(act, cnt, ZERO)
            bound = jnp.where(act, m, bound)
        alpha = jnp.clip(alpha, ALPHA_LO, HALF_PI)

        close = pd <= alpha
        coop = final_mask & close
        comp = final_mask & (~close)

        sm = (jnp.sin(ai) + jnp.sin(aj)) * HALF
        cm = (jnp.cos(ai) + jnp.cos(aj)) * HALF
        mean_ang = jnp.arctan2(sm, cm)
        di = _circ(ai, mean_ang)
        dj = _circ(aj, mean_ang)
        n_c = jnp.sum(coop.astype(_F32))
        coop_sum = jnp.sum(jnp.where(coop, di * di + dj * dj, ZERO))
        coop_term = jnp.where(n_c > ZERO, coop_sum / jnp.maximum(n_c, ONE),
                              ZERO)

        mm = jnp.sum(comp.astype(_F32))
        w1 = jnp.max(jnp.where(comp, nsym, NEG_INF))
        c1 = jnp.sum((comp & (nsym == w1)).astype(_F32))
        rest = jnp.max(jnp.where(comp & (nsym < w1), nsym, NEG_INF))
        w2 = jnp.where(c1 >= TWO, w1, rest)
        margin = jnp.where(mm >= TWO, w2 + (TWO / (mm + ONE)) * (w1 - w2), w1)
        viol = jnp.maximum(nsym - margin, ZERO)
        comp_sum = jnp.sum(jnp.where(comp, viol * viol, ZERO))
        comp_term = jnp.where(mm > ZERO, comp_sum / jnp.maximum(mm, ONE),
                              ZERO)

        valid_b = n > ZERO
        total_coop = total_coop + jnp.where(valid_b, coop_term, ZERO)
        total_comp = total_comp + jnp.where(valid_b, comp_term, ZERO)
        valid = valid + valid_b.astype(_F32)

    denom = jnp.maximum(valid, ONE)
    ones = jnp.full((1, 1), ONE, _F32)
    coop_out[...] = ones * (total_coop / denom)
    comp_out[...] = ones * (total_comp / denom)


def kernel(query_features, predicted_angles, W):
    qf = query_features.astype(jnp.float32)
    pa = predicted_angles.astype(jnp.float32)
    w = W.astype(jnp.float32)
    coop, comp = pl.pallas_call(
        _ogqc_kernel,
        out_shape=(
            jax.ShapeDtypeStruct((1, 1), jnp.float32),
            jax.ShapeDtypeStruct((1, 1), jnp.float32),
        ),
    )(qf, pa, w)
    return (coop[0, 0], comp[0, 0])


# speedup vs baseline: 1.0179x; 1.0179x over previous
import math

import numpy as np
import jax
import jax.numpy as jnp
from jax import lax
from jax.experimental import pallas as pl

NBINS = 36
Q = 300
BATCH = 4
TOPK = 8
ALPHA_ITERS = 24

F = np.float32
NEG_INF = F(-np.inf)
POS_INF = F(np.inf)
TWO_PI = F(2.0 * math.pi)
INV_BINSZ = F(NBINS / (2.0 * math.pi))
HALF_PI = F(math.pi / 2.0)
ALPHA_LO = F(math.pi / NBINS)
QF = F(Q)
INV_QF = F(1.0 / Q)
ZERO = F(0.0)
ONE = F(1.0)
NEG1 = F(-1.0)
HALF = F(0.5)
TWO = F(2.0)
BIG_NEG = F(-1e30)
EPS = F(1e-12)

_F32 = jnp.float32
_HI = lax.Precision.HIGHEST


def _eye_f32(n):
    r = lax.broadcasted_iota(jnp.int32, (n, n), 0)
    c = lax.broadcasted_iota(jnp.int32, (n, n), 1)
    return (r == c).astype(_F32)


def _tp(x, eye):
    return lax.dot_general(x, eye, (((0,), (0,)), ((), ())),
                           preferred_element_type=_F32, precision=_HI)


def _circ(a, b):
    d = jnp.abs(a - b)
    return jnp.minimum(jnp.minimum(d, TWO_PI - d), HALF_PI)


def _ogqc_kernel(qf_ref, pa_ref, w_ref, coop_out, comp_out):
    e300 = _eye_f32(Q)
    e36 = _eye_f32(NBINS)

    ii = lax.broadcasted_iota(jnp.int32, (Q, Q), 0)
    jj = lax.broadcasted_iota(jnp.int32, (Q, Q), 1)
    lower = ii > jj
    row36_f = lax.broadcasted_iota(jnp.int32, (NBINS, 1), 0).astype(_F32)
    block_a = jnp.concatenate(
        [jnp.full((a + 1, 1), F(a), _F32) for a in range(NBINS)], axis=0)
    block_r = jnp.concatenate(
        [row36_f[0:a + 1, :] for a in range(NBINS)], axis=0)
    pad36 = jnp.full((NBINS, 1), ZERO, _F32)
    lane_q = lax.broadcasted_iota(jnp.int32, (1, Q), 1).astype(_F32)
    lane_36f = lax.broadcasted_iota(jnp.int32, (1, NBINS), 1).astype(_F32)
    aa36 = lax.broadcasted_iota(jnp.int32, (NBINS, NBINS), 0)
    cc36 = lax.broadcasted_iota(jnp.int32, (NBINS, NBINS), 1)
    ut36 = aa36 <= cc36
    offdiag36 = aa36 != cc36

    w_mat = w_ref[...]

    total_coop = jnp.full((), ZERO, _F32)
    total_comp = jnp.full((), ZERO, _F32)
    valid = jnp.full((), ZERO, _F32)

    for b in range(BATCH):
        pa = pa_ref[b]
        x_col = pa[:, 0:1]
        y_col = pa[:, 1:2]
        ang_col = jnp.arctan2(y_col, x_col)
        ang_col = jnp.where(ang_col < ZERO, ang_col + TWO_PI, ang_col)
        bins_f_col = jnp.clip(jnp.floor(ang_col * INV_BINSZ), ZERO,
                              F(NBINS - 1))
        bins_f_row = _tp(bins_f_col, e300)
        ang_row = _tp(ang_col, e300)

        onehot = (bins_f_col == lane_36f).astype(_F32)
        fused = qf_ref[b] + lax.dot_general(
            onehot, w_mat, (((1,), (0,)), ((), ())),
            preferred_element_type=_F32, precision=_HI)
        nrm = jnp.sqrt(jnp.sum(fused * fused, axis=1, keepdims=True))
        nq = fused / jnp.maximum(nrm, EPS)
        sim = lax.dot_general(nq, nq, (((1,), (1,)), ((), ())),
                              preferred_element_type=_F32, precision=_HI)

        sim_lo = jnp.where(lower, sim, NEG_INF)

        bound = jnp.full((), POS_INF, _F32)
        cum = jnp.full((), ZERO, _F32)
        kth = jnp.full((), NEG_INF, _F32)
        for _ in range(TOPK // 2):
            act = cum < F(TOPK // 2)
            m = jnp.max(jnp.where(sim_lo < bound, sim_lo, NEG_INF))
            cnt = jnp.sum((sim_lo == m).astype(_F32))
            kth = jnp.where(act, m, kth)
            cum = cum + jnp.where(act, cnt, ZERO)
            bound = jnp.where(act, m, bound)

        m1_rows = []
        for c in range(NBINS):
            v = jnp.where(bins_f_col == F(c), sim_lo, NEG_INF)
            m1_rows.append(jnp.max(v, axis=0, keepdims=True))
        m1t = jnp.concatenate(m1_rows, axis=0)

        nh_cols = []
        for a in range(NBINS):
            amask = bins_f_row == F(a)
            nh_cols.append(jnp.max(jnp.where(amask, m1t, NEG_INF), axis=1,
                                   keepdims=True))
        nhalf = jnp.maximum(jnp.concatenate(nh_cols, axis=1), BIG_NEG)
        nsym = jnp.maximum(nhalf, _tp(nhalf, e36))

        ch_cols = []
        for a in range(NBINS):
            cond = (bins_f_row == F(a)) & (m1t == nsym[:, a:a + 1])
            ch_cols.append(jnp.max(jnp.where(cond, lane_q, NEG1), axis=1,
                                   keepdims=True))
        codehalf = jnp.concatenate(ch_cols, axis=1)
        sel_i = jnp.maximum(codehalf, _tp(codehalf, e36))

        sim_ext = jnp.concatenate([sim, bins_f_col, ang_col], axis=1)
        si_all = jnp.concatenate(
            [sel_i[0:a + 1, a:a + 1] for a in range(NBINS)], axis=0)
        nsym_all = jnp.concatenate(
            [nsym[0:a + 1, a:a + 1] for a in range(NBINS)], axis=0)
        oh_all = (si_all == lane_q).astype(_F32)
        r_ext = lax.dot_general(oh_all, sim_ext, (((1,), (0,)), ((), ())),
                                preferred_element_type=_F32, precision=_HI)
        r_sim = r_ext[:, 0:Q]
        b_sel = r_ext[:, Q:Q + 1]
        ai_all = r_ext[:, Q + 1:Q + 2]
        jbin_all = jnp.where(b_sel == block_a, block_r, block_a)
        maskj = ((bins_f_row == jbin_all) & (lane_q > si_all)
                 & (r_sim == nsym_all))
        sj_all = jnp.max(jnp.where(maskj, lane_q, NEG1), axis=1,
                         keepdims=True)
        aj_all = jnp.sum(jnp.where(sj_all == lane_q, ang_row, ZERO), axis=1,
                         keepdims=True)
        ai_cols = []
        aj_cols = []
        off = 0
        for a in range(NBINS):
            h = a + 1
            if a < NBINS - 1:
                ai_cols.append(jnp.concatenate(
                    [ai_all[off:off + h, :], pad36[0:NBINS - h, :]], axis=0))
                aj_cols.append(jnp.concatenate(
                    [aj_all[off:off + h, :], pad36[0:NBINS - h, :]], axis=0))
            else:
                ai_cols.append(ai_all[off:off + h, :])
                aj_cols.append(aj_all[off:off + h, :])
            off += h
        ai = jnp.concatenate(ai_cols, axis=1)
        aj = jnp.concatenate(aj_cols, axis=1)

        hist_row = jnp.sum(onehot, axis=0, keepdims=True)
        hist_col = lax.dot_general(e36, hist_row, (((1,), (1,)), ((), ())),
                                   preferred_element_type=_F32, precision=_HI)
        present = ((hist_col >= ONE) & (hist_row >= ONE)
                   & (offdiag36 | (hist_col >= TWO)))
        final_mask = present & ut36 & (nsym >= kth)
        n = jnp.sum(final_mask.astype(_F32))

        pd = _circ(ai, aj)
        pdm = jnp.where(final_mask, pd, POS_INF)
        target = jnp.floor(jnp.maximum(n - ONE, ZERO) * HALF) + ONE
        bound = jnp.full((), NEG_INF, _F32)
        cum = jnp.full((), ZERO, _F32)
        alpha = jnp.full((), POS_INF, _F32)
        for _ in range(ALPHA_ITERS):
            act = cum < target
            m = jnp.min(jnp.where(pdm > bound, pdm, POS_INF))
            cnt = jnp.sum((pdm == m).astype(_F32))
            alpha = jnp.where(act, m, alpha)
            cum = cum + jnp.where(act, cnt, ZERO)
            bound = jnp.where(act, m, bound)
        alpha = jnp.clip(alpha, ALPHA_LO, HALF_PI)

        close = pd <= alpha
        coop = final_mask & close
        comp = final_mask & (~close)

        sm = (jnp.sin(ai) + jnp.sin(aj)) * HALF
        cm = (jnp.cos(ai) + jnp.cos(aj)) * HALF
        mean_ang = jnp.arctan2(sm, cm)
        di = _circ(ai, mean_ang)
        dj = _circ(aj, mean_ang)
        n_c = jnp.sum(coop.astype(_F32))
        coop_sum = jnp.sum(jnp.where(coop, di * di + dj * dj, ZERO))
        coop_term = jnp.where(n_c > ZERO, coop_sum / jnp.maximum(n_c, ONE),
                              ZERO)

        mm = jnp.sum(comp.astype(_F32))
        w1 = jnp.max(jnp.where(comp, nsym, NEG_INF))
        c1 = jnp.sum((comp & (nsym == w1)).astype(_F32))
        rest = jnp.max(jnp.where(comp & (nsym < w1), nsym, NEG_INF))
        w2 = jnp.where(c1 >= TWO, w1, rest)
        margin = jnp.where(mm >= TWO, w2 + (TWO / (mm + ONE)) * (w1 - w2), w1)
        viol = jnp.maximum(nsym - margin, ZERO)
        comp_sum = jnp.sum(jnp.where(comp, viol * viol, ZERO))
        comp_term = jnp.where(mm > ZERO, comp_sum / jnp.maximum(mm, ONE),
                              ZERO)

        valid_b = n > ZERO
        total_coop = total_coop + jnp.where(valid_b, coop_term, ZERO)
        total_comp = total_comp + jnp.where(valid_b, comp_term, ZERO)
        valid = valid + valid_b.astype(_F32)

    denom = jnp.maximum(valid, ONE)
    ones = jnp.full((1, 1), ONE, _F32)
    coop_out[...] = ones * (total_coop / denom)
    comp_out[...] = ones * (total_comp / denom)


def kernel(query_features, predicted_angles, W):
    qf = query_features.astype(jnp.float32)
    pa = predicted_angles.astype(jnp.float32)
    w = W.astype(jnp.float32)
    coop, comp = pl.pallas_call(
        _ogqc_kernel,
        out_shape=(
            jax.ShapeDtypeStruct((1, 1), jnp.float32),
            jax.ShapeDtypeStruct((1, 1), jnp.float32),
        ),
    )(qf, pa, w)
    return (coop[0, 0], comp[0, 0])


# speedup vs baseline: 1.1192x; 1.0995x over previous
import math

import numpy as np
import jax
import jax.numpy as jnp
from jax import lax
from jax.experimental import pallas as pl

NBINS = 36
Q = 300
BATCH = 4
TOPK = 8
ALPHA_ITERS = 24

F = np.float32
NEG_INF = F(-np.inf)
POS_INF = F(np.inf)
TWO_PI = F(2.0 * math.pi)
INV_BINSZ = F(NBINS / (2.0 * math.pi))
HALF_PI = F(math.pi / 2.0)
ALPHA_LO = F(math.pi / NBINS)
QF = F(Q)
INV_QF = F(1.0 / Q)
ZERO = F(0.0)
ONE = F(1.0)
NEG1 = F(-1.0)
HALF = F(0.5)
TWO = F(2.0)
BIG_NEG = F(-1e30)
EPS = F(1e-12)

_F32 = jnp.float32
_HI = lax.Precision.HIGHEST


def _eye_f32(n):
    r = lax.broadcasted_iota(jnp.int32, (n, n), 0)
    c = lax.broadcasted_iota(jnp.int32, (n, n), 1)
    return (r == c).astype(_F32)


def _tp(x, eye):
    return lax.dot_general(x, eye, (((0,), (0,)), ((), ())),
                           preferred_element_type=_F32, precision=_HI)


def _circ(a, b):
    d = jnp.abs(a - b)
    return jnp.minimum(jnp.minimum(d, TWO_PI - d), HALF_PI)


def _ogqc_kernel(qf_ref, pa_ref, w_ref, coop_out, comp_out):
    e300 = _eye_f32(Q)
    e36 = _eye_f32(NBINS)

    ii = lax.broadcasted_iota(jnp.int32, (Q, Q), 0)
    jj = lax.broadcasted_iota(jnp.int32, (Q, Q), 1)
    lower = ii > jj
    row36_f = lax.broadcasted_iota(jnp.int32, (NBINS, 1), 0).astype(_F32)
    block_a = jnp.concatenate(
        [jnp.full((a + 1, 1), F(a), _F32) for a in range(NBINS)], axis=0)
    block_r = jnp.concatenate(
        [row36_f[0:a + 1, :] for a in range(NBINS)], axis=0)
    pad36 = jnp.full((NBINS, 1), ZERO, _F32)
    lane_q = lax.broadcasted_iota(jnp.int32, (1, Q), 1).astype(_F32)
    lane_36f = lax.broadcasted_iota(jnp.int32, (1, NBINS), 1).astype(_F32)
    aa36 = lax.broadcasted_iota(jnp.int32, (NBINS, NBINS), 0)
    cc36 = lax.broadcasted_iota(jnp.int32, (NBINS, NBINS), 1)
    ut36 = aa36 <= cc36
    offdiag36 = aa36 != cc36

    w_mat = w_ref[...]

    total_coop = jnp.full((), ZERO, _F32)
    total_comp = jnp.full((), ZERO, _F32)
    valid = jnp.full((), ZERO, _F32)

    for b in range(BATCH):
        pa = pa_ref[b]
        x_col = pa[:, 0:1]
        y_col = pa[:, 1:2]
        ang_col = jnp.arctan2(y_col, x_col)
        ang_col = jnp.where(ang_col < ZERO, ang_col + TWO_PI, ang_col)
        bins_f_col = jnp.clip(jnp.floor(ang_col * INV_BINSZ), ZERO,
                              F(NBINS - 1))
        bins_f_row = _tp(bins_f_col, e300)

        onehot = (bins_f_col == lane_36f).astype(_F32)
        fused = qf_ref[b] + lax.dot_general(
            onehot, w_mat, (((1,), (0,)), ((), ())),
            preferred_element_type=_F32, precision=_HI)
        nrm = jnp.sqrt(jnp.sum(fused * fused, axis=1, keepdims=True))
        nq = fused / jnp.maximum(nrm, EPS)
        sim = lax.dot_general(nq, nq, (((1,), (1,)), ((), ())),
                              preferred_element_type=_F32, precision=_HI)

        sim_lo = jnp.where(lower, sim, NEG_INF)

        bound = jnp.full((), POS_INF, _F32)
        cum = jnp.full((), ZERO, _F32)
        kth = jnp.full((), NEG_INF, _F32)
        for _ in range(TOPK // 2):
            act = cum < F(TOPK // 2)
            m = jnp.max(jnp.where(sim_lo < bound, sim_lo, NEG_INF))
            cnt = jnp.sum((sim_lo == m).astype(_F32))
            kth = jnp.where(act, m, kth)
            cum = cum + jnp.where(act, cnt, ZERO)
            bound = jnp.where(act, m, bound)

        m1_rows = []
        for c in range(NBINS):
            v = jnp.where(bins_f_col == F(c), sim_lo, NEG_INF)
            m1_rows.append(jnp.max(v, axis=0, keepdims=True))
        m1t = jnp.concatenate(m1_rows, axis=0)

        nh_cols = []
        for a in range(NBINS):
            amask = bins_f_row == F(a)
            nh_cols.append(jnp.max(jnp.where(amask, m1t, NEG_INF), axis=1,
                                   keepdims=True))
        nhalf = jnp.maximum(jnp.concatenate(nh_cols, axis=1), BIG_NEG)
        nsym = jnp.maximum(nhalf, _tp(nhalf, e36))

        ch_cols = []
        for a in range(NBINS):
            cond = (bins_f_row == F(a)) & (m1t == nsym[:, a:a + 1])
            ch_cols.append(jnp.max(jnp.where(cond, lane_q, NEG1), axis=1,
                                   keepdims=True))
        codehalf = jnp.concatenate(ch_cols, axis=1)
        sel_i = jnp.maximum(codehalf, _tp(codehalf, e36))

        sim_ext = jnp.concatenate([sim, bins_f_col, ang_col], axis=1)
        si_all = jnp.concatenate(
            [sel_i[0:a + 1, a:a + 1] for a in range(NBINS)], axis=0)
        nsym_all = jnp.concatenate(
            [nsym[0:a + 1, a:a + 1] for a in range(NBINS)], axis=0)
        oh_all = (si_all == lane_q).astype(_F32)
        r_ext = lax.dot_general(oh_all, sim_ext, (((1,), (0,)), ((), ())),
                                preferred_element_type=_F32, precision=_HI)
        r_sim = r_ext[:, 0:Q]
        b_sel = r_ext[:, Q:Q + 1]
        ai_all = r_ext[:, Q + 1:Q + 2]
        jbin_all = jnp.where(b_sel == block_a, block_r, block_a)
        maskj = ((bins_f_row == jbin_all) & (lane_q > si_all)
                 & (r_sim == nsym_all))
        sj_all = jnp.max(jnp.where(maskj, lane_q, NEG1), axis=1,
                         keepdims=True)
        ohj_all = (sj_all == lane_q).astype(_F32)
        aj_all = lax.dot_general(ohj_all, ang_col, (((1,), (0,)), ((), ())),
                                 preferred_element_type=_F32, precision=_HI)
        ai_cols = []
        aj_cols = []
        off = 0
        for a in range(NBINS):
            h = a + 1
            if a < NBINS - 1:
                ai_cols.append(jnp.concatenate(
                    [ai_all[off:off + h, :], pad36[0:NBINS - h, :]], axis=0))
                aj_cols.append(jnp.concatenate(
                    [aj_all[off:off + h, :], pad36[0:NBINS - h, :]], axis=0))
            else:
                ai_cols.append(ai_all[off:off + h, :])
                aj_cols.append(aj_all[off:off + h, :])
            off += h
        ai = jnp.concatenate(ai_cols, axis=1)
        aj = jnp.concatenate(aj_cols, axis=1)

        hist_row = jnp.sum(onehot, axis=0, keepdims=True)
        hist_col = lax.dot_general(e36, hist_row, (((1,), (1,)), ((), ())),
                                   preferred_element_type=_F32, precision=_HI)
        present = ((hist_col >= ONE) & (hist_row >= ONE)
                   & (offdiag36 | (hist_col >= TWO)))
        final_mask = present & ut36 & (nsym >= kth)
        n = jnp.sum(final_mask.astype(_F32))

        pd = _circ(ai, aj)
        pdm = jnp.where(final_mask, pd, POS_INF)
        target = jnp.floor(jnp.maximum(n - ONE, ZERO) * HALF) + ONE
        bound = jnp.full((), NEG_INF, _F32)
        cum = jnp.full((), ZERO, _F32)
        alpha = jnp.full((), POS_INF, _F32)
        for _ in range(ALPHA_ITERS):
            act = cum < target
            m = jnp.min(jnp.where(pdm > bound, pdm, POS_INF))
            cnt = jnp.sum((pdm == m).astype(_F32))
            alpha = jnp.where(act, m, alpha)
            cum = cum + jnp.where(act, cnt, ZERO)
            bound = jnp.where(act, m, bound)
        alpha = jnp.clip(alpha, ALPHA_LO, HALF_PI)

        close = pd <= alpha
        coop = final_mask & close
        comp = final_mask & (~close)

        sm = (jnp.sin(ai) + jnp.sin(aj)) * HALF
        cm = (jnp.cos(ai) + jnp.cos(aj)) * HALF
        mean_ang = jnp.arctan2(sm, cm)
        di = _circ(ai, mean_ang)
        dj = _circ(aj, mean_ang)
        n_c = jnp.sum(coop.astype(_F32))
        coop_sum = jnp.sum(jnp.where(coop, di * di + dj * dj, ZERO))
        coop_term = jnp.where(n_c > ZERO, coop_sum / jnp.maximum(n_c, ONE),
                              ZERO)

        mm = jnp.sum(comp.astype(_F32))
        w1 = jnp.max(jnp.where(comp, nsym, NEG_INF))
        c1 = jnp.sum((comp & (nsym == w1)).astype(_F32))
        rest = jnp.max(jnp.where(comp & (nsym < w1), nsym, NEG_INF))
        w2 = jnp.where(c1 >= TWO, w1, rest)
        margin = jnp.where(mm >= TWO, w2 + (TWO / (mm + ONE)) * (w1 - w2), w1)
        viol = jnp.maximum(nsym - margin, ZERO)
        comp_sum = jnp.sum(jnp.where(comp, viol * viol, ZERO))
        comp_term = jnp.where(mm > ZERO, comp_sum / jnp.maximum(mm, ONE),
                              ZERO)

        valid_b = n > ZERO
        total_coop = total_coop + jnp.where(valid_b, coop_term, ZERO)
        total_comp = total_comp + jnp.where(valid_b, comp_term, ZERO)
        valid = valid + valid_b.astype(_F32)

    denom = jnp.maximum(valid, ONE)
    ones = jnp.full((1, 1), ONE, _F32)
    coop_out[...] = ones * (total_coop / denom)
    comp_out[...] = ones * (total_comp / denom)


def kernel(query_features, predicted_angles, W):
    qf = query_features.astype(jnp.float32)
    pa = predicted_angles.astype(jnp.float32)
    w = W.astype(jnp.float32)
    coop, comp = pl.pallas_call(
        _ogqc_kernel,
        out_shape=(
            jax.ShapeDtypeStruct((1, 1), jnp.float32),
            jax.ShapeDtypeStruct((1, 1), jnp.float32),
        ),
    )(qf, pa, w)
    return (coop[0, 0], comp[0, 0])
